# Initial kernel scaffold; baseline (speedup 1.0000x reference)
import jax
import jax.numpy as jnp
from jax.experimental import pallas as pl


def kernel(A_hat_indices, A_hat_values, o_embedding, u_id, user_table, W1, b1, W2, b2):
    raise NotImplementedError("write your pallas kernel here")


# speedup vs baseline: 13.2284x; 13.2284x over previous
import functools

import jax
import jax.numpy as jnp
from jax import lax
from jax.experimental import pallas as pl
from jax.experimental.pallas import tpu as pltpu
from jax.experimental.pallas import tpu_sc as plsc

N = 16384
D = 64
L = 3
NNZ = 2621440
NC = 2
NS = 16
NW = NC * NS
E_TILE = NNZ // NW
CHUNK = 128
GRP = 8
N_CHUNKS = E_TILE // CHUNK
N_GRPS = N_CHUNKS // GRP
ROWS_TILE = N // NS

_sc_mesh = plsc.VectorSubcoreMesh(core_axis_name="c", subcore_axis_name="s")


@functools.partial(
    pl.kernel,
    out_type=jax.ShapeDtypeStruct((NC, N, D), jnp.float32),
    mesh=_sc_mesh,
    compiler_params=pltpu.CompilerParams(use_tc_tiling_on_sc=False),
    scratch_types=[
        pltpu.VMEM((GRP, CHUNK), jnp.int32),
        pltpu.VMEM((GRP, CHUNK), jnp.int32),
        pltpu.VMEM((GRP, CHUNK), jnp.float32),
        pltpu.VMEM((CHUNK, D), jnp.float32),
        pltpu.VMEM((CHUNK, D), jnp.float32),
        pltpu.VMEM_SHARED((N, D), jnp.float32),
        pltpu.SemaphoreType.DMA,
    ],
)
def _sc_spmm(ego_hbm, col_hbm, row_hbm, val_hbm, zeros_hbm, out_hbm,
             colbuf, rowbuf, valbuf, rows, scaled, acc, sem):
    c = lax.axis_index("c")
    s = lax.axis_index("s")
    tile_chunk_base = (c * NS + s) * N_CHUNKS

    pltpu.sync_copy(zeros_hbm, acc.at[pl.ds(s * ROWS_TILE, ROWS_TILE)])
    plsc.subcore_barrier()

    def do_group(g, _):
        gbase = tile_chunk_base + g * GRP
        pltpu.sync_copy(col_hbm.at[pl.ds(gbase, GRP)], colbuf)
        pltpu.sync_copy(row_hbm.at[pl.ds(gbase, GRP)], rowbuf)
        pltpu.sync_copy(val_hbm.at[pl.ds(gbase, GRP)], valbuf)

        def do_chunk(k, _):
            pltpu.async_copy(ego_hbm.at[colbuf.at[k]], rows, sem).wait()

            def scale16(j, _):
                vv = valbuf[k, pl.ds(j * 16, 16)]
                for i in range(16):
                    e = j * 16 + i
                    v = vv[i]
                    for q in range(D // 16):
                        scaled[e, pl.ds(q * 16, 16)] = (
                            rows[e, pl.ds(q * 16, 16)] * v)
                return 0

            lax.fori_loop(0, CHUNK // 16, scale16, 0)
            pltpu.sync_copy(scaled, acc.at[rowbuf.at[k]], add=True)
            return 0

        lax.fori_loop(0, GRP, do_chunk, 0)
        return 0

    lax.fori_loop(0, N_GRPS, do_group, 0)

    plsc.subcore_barrier()
    pltpu.sync_copy(acc.at[pl.ds(s * ROWS_TILE, ROWS_TILE)],
                    out_hbm.at[c, pl.ds(s * ROWS_TILE, ROWS_TILE)])


RBLK = 2048


def _tc_body(p0, p1, ego, all_in, w1, b1, w2, b2, ego_out, all_out):
    side = p0[...] + p1[...]
    e = ego[...]
    dn = (((1,), (1,)), ((), ()))
    sum_emb = lax.dot_general(side, w1[...], dn,
                              preferred_element_type=jnp.float32) + b1[...]
    bi = lax.dot_general(e * side, w2[...], dn,
                         preferred_element_type=jnp.float32) + b2[...]
    x = sum_emb + bi
    act = jnp.where(x >= 0, x, 0.01 * x)
    nrm = jnp.sqrt(jnp.sum(act * act, axis=1, keepdims=True))
    norm = act / jnp.maximum(nrm, 1e-12)
    ego_out[...] = act
    all_out[...] = all_in[...] + norm


def _tc_dense(p0, p1, ego, all_in, w1, b1, w2, b2):
    row_spec = pl.BlockSpec((RBLK, D), lambda i: (i, 0))
    mat_spec = pl.BlockSpec((D, D), lambda i: (0, 0))
    vec_spec = pl.BlockSpec((1, D), lambda i: (0, 0))
    return pl.pallas_call(
        _tc_body,
        grid=(N // RBLK,),
        in_specs=[row_spec, row_spec, row_spec, row_spec,
                  mat_spec, vec_spec, mat_spec, vec_spec],
        out_specs=[row_spec, row_spec],
        out_shape=[jax.ShapeDtypeStruct((N, D), jnp.float32),
                   jax.ShapeDtypeStruct((N, D), jnp.float32)],
    )(p0, p1, ego, all_in, w1, b1, w2, b2)


def kernel(A_hat_indices, A_hat_values, o_embedding, u_id, user_table, W1, b1, W2, b2):
    row = A_hat_indices[0]
    col = A_hat_indices[1]
    ego = jnp.concatenate([jnp.take(user_table, u_id, axis=0), o_embedding], axis=0)
    all_emb = ego
    col2d = col.reshape(NNZ // CHUNK, CHUNK)
    row2d = row.reshape(NNZ // CHUNK, CHUNK)
    val2d = A_hat_values.reshape(NNZ // CHUNK, CHUNK)
    zeros = jnp.zeros((ROWS_TILE, D), jnp.float32)
    for i in range(L):
        parts = _sc_spmm(ego, col2d, row2d, val2d, zeros)
        ego, all_emb = _tc_dense(parts[0], parts[1], ego, all_emb,
                                 W1[i], b1[i].reshape(1, D),
                                 W2[i], b2[i].reshape(1, D))
    return all_emb
